# Initial kernel scaffold; baseline (speedup 1.0000x reference)
import jax
import jax.numpy as jnp
from jax.experimental import pallas as pl


def kernel(token_ids, weight):
    raise NotImplementedError("write your pallas kernel here")


# speedup vs baseline: 1.6859x; 1.6859x over previous
import functools

import jax
import jax.numpy as jnp
from jax import lax
from jax.experimental import pallas as pl
from jax.experimental.pallas import tpu as pltpu
from jax.experimental.pallas import tpu_sc as plsc

NUM_EMB = 1000000
DIM = 64
BATCH = 16384
HIST = 50

NC = 2
NS = 16
NW = NC * NS
TOTAL = BATCH * HIST
CHUNK = 128
PER_W = TOTAL // NW
NCHUNK = PER_W // CHUNK

_MESH = plsc.VectorSubcoreMesh(core_axis_name="c", subcore_axis_name="s")


@functools.partial(
    pl.kernel,
    mesh=_MESH,
    out_type=jax.ShapeDtypeStruct((TOTAL, DIM), jnp.float32),
    scratch_types=[
        pltpu.VMEM((NCHUNK, CHUNK), jnp.int32),
        pltpu.VMEM((CHUNK, DIM), jnp.float32),
        pltpu.SemaphoreType.DMA,
    ],
    compiler_params=pltpu.CompilerParams(use_tc_tiling_on_sc=False),
)
def _emb_lookup(idx_hbm, tbl_hbm, out_hbm, idx_v, rows_v, gsem):
    wid = lax.axis_index("s") * NC + lax.axis_index("c")
    base = wid * PER_W
    pltpu.sync_copy(idx_hbm.at[wid], idx_v)

    def body(j, carry):
        pltpu.async_copy(tbl_hbm.at[idx_v.at[j]], rows_v, gsem).wait()
        pltpu.sync_copy(rows_v, out_hbm.at[pl.ds(base + j * CHUNK, CHUNK)])
        return carry

    lax.fori_loop(0, NCHUNK, body, 0)


def kernel(token_ids, weight):
    idx = token_ids.astype(jnp.int32).reshape(NW, NCHUNK, CHUNK)
    out = _emb_lookup(idx, weight)
    return out.reshape(BATCH, HIST, DIM)


# speedup vs baseline: 1.8292x; 1.0850x over previous
import functools

import jax
import jax.numpy as jnp
from jax import lax
from jax.experimental import pallas as pl
from jax.experimental.pallas import tpu as pltpu
from jax.experimental.pallas import tpu_sc as plsc

NUM_EMB = 1000000
DIM = 64
BATCH = 16384
HIST = 50

NC = 2
NS = 16
NW = NC * NS
TOTAL = BATCH * HIST
CHUNK = 512
PER_W = TOTAL // NW
NCHUNK = PER_W // CHUNK

_MESH = plsc.VectorSubcoreMesh(core_axis_name="c", subcore_axis_name="s")


@functools.partial(
    pl.kernel,
    mesh=_MESH,
    out_type=jax.ShapeDtypeStruct((TOTAL, DIM), jnp.float32),
    scratch_types=[
        pltpu.VMEM((NCHUNK, CHUNK), jnp.int32),
        pltpu.VMEM((CHUNK, DIM), jnp.float32),
        pltpu.SemaphoreType.DMA,
    ],
    compiler_params=pltpu.CompilerParams(use_tc_tiling_on_sc=False),
)
def _emb_lookup(idx_hbm, tbl_hbm, out_hbm, idx_v, rows_v, gsem):
    wid = lax.axis_index("s") * NC + lax.axis_index("c")
    base = wid * PER_W
    pltpu.sync_copy(idx_hbm.at[wid], idx_v)

    def body(j, carry):
        pltpu.async_copy(tbl_hbm.at[idx_v.at[j]], rows_v, gsem).wait()
        pltpu.sync_copy(rows_v, out_hbm.at[pl.ds(base + j * CHUNK, CHUNK)])
        return carry

    lax.fori_loop(0, NCHUNK, body, 0)


def kernel(token_ids, weight):
    idx = token_ids.astype(jnp.int32).reshape(NW, NCHUNK, CHUNK)
    out = _emb_lookup(idx, weight)
    return out.reshape(BATCH, HIST, DIM)


# speedup vs baseline: 1.8775x; 1.0265x over previous
import functools

import jax
import jax.numpy as jnp
from jax import lax
from jax.experimental import pallas as pl
from jax.experimental.pallas import tpu as pltpu
from jax.experimental.pallas import tpu_sc as plsc

NUM_EMB = 1000000
DIM = 64
BATCH = 16384
HIST = 50

NC = 2
NS = 16
NW = NC * NS
TOTAL = BATCH * HIST
CHUNK = 512
PER_W = TOTAL // NW
NCHUNK = PER_W // CHUNK
NBUF = 3

_MESH = plsc.VectorSubcoreMesh(core_axis_name="c", subcore_axis_name="s")


@functools.partial(
    pl.kernel,
    mesh=_MESH,
    out_type=jax.ShapeDtypeStruct((TOTAL, DIM), jnp.float32),
    scratch_types=[
        pltpu.VMEM((NCHUNK, CHUNK), jnp.int32),
        pltpu.VMEM((CHUNK, DIM), jnp.float32),
        pltpu.VMEM((CHUNK, DIM), jnp.float32),
        pltpu.VMEM((CHUNK, DIM), jnp.float32),
        pltpu.SemaphoreType.DMA,
        pltpu.SemaphoreType.DMA,
        pltpu.SemaphoreType.DMA,
        pltpu.SemaphoreType.DMA,
        pltpu.SemaphoreType.DMA,
        pltpu.SemaphoreType.DMA,
    ],
    compiler_params=pltpu.CompilerParams(use_tc_tiling_on_sc=False),
)
def _emb_lookup(idx_hbm, tbl_hbm, out_hbm, idx_v, r0, r1, r2,
                g0, g1, g2, w0, w1, w2):
    wid = lax.axis_index("s") * NC + lax.axis_index("c")
    base = wid * PER_W
    bufs, gs, ws = (r0, r1, r2), (g0, g1, g2), (w0, w1, w2)

    pltpu.sync_copy(idx_hbm.at[wid], idx_v)

    def fire_g(c, p):
        pltpu.make_async_copy(tbl_hbm.at[idx_v.at[c]], bufs[p], gs[p]).start()

    def drain_g(p):
        pltpu.make_async_copy(tbl_hbm.at[idx_v.at[0]], bufs[p], gs[p]).wait()

    def fire_w(c, p):
        pltpu.make_async_copy(
            bufs[p], out_hbm.at[pl.ds(base + c * CHUNK, CHUNK)], ws[p]).start()

    def drain_w(p):
        pltpu.make_async_copy(
            bufs[p], out_hbm.at[pl.ds(base, CHUNK)], ws[p]).wait()

    def phase(c, i):
        drain_g(i)
        fire_w(c, i)
        q = (i + 2) % NBUF
        drain_w(q)
        fire_g(c + 2, q)

    fire_g(0, 0)
    fire_g(1, 1)
    drain_g(0)
    fire_w(0, 0)
    fire_g(2, 2)
    drain_g(1)
    fire_w(1, 1)
    drain_w(0)
    fire_g(3, 0)
    phase(2, 2)

    def body(t, carry):
        phase(3 * t, 0)
        phase(3 * t + 1, 1)
        phase(3 * t + 2, 2)
        return carry

    lax.fori_loop(1, NCHUNK // NBUF, body, 0)

    drain_g(0)
    fire_w(48, 0)
    drain_g(1)
    fire_w(49, 1)
    drain_w(2)
    drain_w(0)
    drain_w(1)


def kernel(token_ids, weight):
    idx = token_ids.astype(jnp.int32).reshape(NW, NCHUNK, CHUNK)
    out = _emb_lookup(idx, weight)
    return out.reshape(BATCH, HIST, DIM)
